# baseline (speedup 1.0000x reference)
import functools

import jax
import jax.numpy as jnp
from jax import lax
from jax.experimental import pallas as pl
from jax.experimental.pallas import tpu as pltpu
from jax.experimental.pallas import tpu_sc as plsc

H = 128
B = 1024
L = 200
BL = B * L
EPS = 1e-12

NC = 2
NS = 16
NW = NC * NS
TOK_PER_W = BL // NW
C = 80
N_CHUNKS = TOK_PER_W // C
N_PAIRS = N_CHUNKS // 2
NV = H // 16


def _rsqrt(x):
    i = lax.bitcast_convert_type(x, jnp.int32)
    i = jnp.int32(0x5F3759DF) - lax.shift_right_logical(i, 1)
    y = lax.bitcast_convert_type(i, jnp.float32)
    for _ in range(3):
        y = y * (1.5 - 0.5 * x * y * y)
    return y


_GDN = lax.GatherDimensionNumbers(
    offset_dims=(), collapsed_slice_dims=(0,), start_index_map=(0,))


def _perm(v, idx):
    return lax.gather(v, idx[:, None], _GDN, (1,),
                      mode=lax.GatherScatterMode.PROMISE_IN_BOUNDS)


def _hsum(v):
    idx = lax.iota(jnp.int32, 16)
    for d in (8, 4, 2, 1):
        v = v + _perm(v, idx ^ d)
    return v


_MESH = plsc.VectorSubcoreMesh(
    core_axis_name="c", subcore_axis_name="s", num_cores=NC, num_subcores=NS
)


@functools.partial(
    pl.kernel,
    out_type=jax.ShapeDtypeStruct((BL, H), jnp.float32),
    mesh=_MESH,
    scratch_types=(
        [pltpu.VMEM((6, C), jnp.int32) for _ in range(2)]
        + [pltpu.VMEM((4, C, H), jnp.float32) for _ in range(2)]
        + [pltpu.VMEM((C, H), jnp.float32) for _ in range(2)]
        + [pltpu.VMEM((2, H), jnp.float32),
           pltpu.VMEM((10, H), jnp.float32),
           pltpu.VMEM((20, H), jnp.float32),
           pltpu.VMEM((120, H), jnp.float32),
           pltpu.VMEM((H,), jnp.float32), pltpu.VMEM((H,), jnp.float32),
           pltpu.SemaphoreType.DMA, pltpu.SemaphoreType.DMA,
           pltpu.SemaphoreType.DMA, pltpu.SemaphoreType.DMA]
    ),
)
def _embed_ln(ids3, wt, nt, dt, pt, st, mt, at, g, b,
              out,
              idx_a, idx_b, rows_a, rows_b, ob_a, ob_b,
              seg_v, mod_v, sm_v, age_v, gv, bv,
              sem_a, sem_b, osem_a, osem_b):
    wid = lax.axis_index("c") * NS + lax.axis_index("s")
    chunk0 = wid * N_CHUNKS
    tok0 = wid * TOK_PER_W

    pltpu.sync_copy(st, seg_v)
    pltpu.sync_copy(mt, mod_v)
    pltpu.sync_copy(at, age_v)
    pltpu.sync_copy(g, gv)
    pltpu.sync_copy(b, bv)
    gs = [gv[pl.ds(k * 16, 16)] for k in range(NV)]
    bs = [bv[pl.ds(k * 16, 16)] for k in range(NV)]

    def build_sm(sm, c2):
        s_i = sm // 10
        m_i = sm - s_i * 10
        for k in range(NV):
            sl = pl.ds(k * 16, 16)
            sm_v[sm, sl] = seg_v[s_i, sl] + mod_v[m_i, sl]
        return c2
    lax.fori_loop(0, 20, build_sm, 0)

    htabs = (wt, nt, dt, pt)
    rot8 = lax.iota(jnp.int32, 16) ^ 8

    def fire_gather(idx, rows, sem, ci):
        pltpu.sync_copy(ids3.at[chunk0 + ci], idx)
        for ti in range(4):
            pltpu.async_copy(htabs[ti].at[idx.at[ti]], rows.at[ti], sem)

    def wait_gather(idx, rows, sem):
        for ti in range(4):
            pltpu.make_async_copy(htabs[ti].at[idx.at[ti]], rows.at[ti],
                                  sem).wait()

    def compute_token(rows, ob, t, id_sm, id_a):
        vs = []
        for k in range(NV):
            sl = pl.ds(k * 16, 16)
            v = ((rows[0, t, sl] + rows[1, t, sl])
                 + (rows[2, t, sl] + rows[3, t, sl])
                 + (sm_v[id_sm, sl] + age_v[id_a, sl]))
            vs.append(v)
        s = ((vs[0] + vs[1]) + (vs[2] + vs[3])) + (
            (vs[4] + vs[5]) + (vs[6] + vs[7]))
        sq = ((vs[0] * vs[0] + vs[1] * vs[1])
              + (vs[2] * vs[2] + vs[3] * vs[3])) + (
             (vs[4] * vs[4] + vs[5] * vs[5])
              + (vs[6] * vs[6] + vs[7] * vs[7]))
        u = _hsum(s) * (1.0 / H)
        ex2 = _hsum(sq) * (1.0 / H)
        var = jnp.maximum(ex2 - u * u, 0.0)
        inv = _rsqrt(var + EPS)
        for k in range(NV):
            ob[t, pl.ds(k * 16, 16)] = (vs[k] - u) * inv * gs[k] + bs[k]

    def compute_chunk(idx, rows, ob, osem, ci):
        def group_body(g16, c2):
            base16 = g16 * 16
            ids_sm0 = idx[4, pl.ds(base16, 16)]
            ids_a0 = idx[5, pl.ds(base16, 16)]

            for u in range(16):
                compute_token(rows, ob, base16 + u, ids_sm0[u], ids_a0[u])
            return c2
        lax.fori_loop(0, C // 16, group_body, 0)
        pltpu.async_copy(ob, out.at[pl.ds(tok0 + ci * C, C)], osem)

    def wait_out(ob, osem, ci):
        pltpu.make_async_copy(ob, out.at[pl.ds(tok0 + ci * C, C)],
                              osem).wait()

    fire_gather(idx_a, rows_a, sem_a, 0)

    def pair_body(p, carry):
        ca = 2 * p
        cb = 2 * p + 1
        fire_gather(idx_b, rows_b, sem_b, cb)
        wait_gather(idx_a, rows_a, sem_a)

        @pl.when(p > 0)
        def _():
            wait_out(ob_a, osem_a, ca - 2)
        compute_chunk(idx_a, rows_a, ob_a, osem_a, ca)

        @pl.when(p < N_PAIRS - 1)
        def _():
            fire_gather(idx_a, rows_a, sem_a, ca + 2)

        wait_gather(idx_b, rows_b, sem_b)

        @pl.when(p > 0)
        def _():
            wait_out(ob_b, osem_b, cb - 2)
        compute_chunk(idx_b, rows_b, ob_b, osem_b, cb)
        return carry

    lax.fori_loop(0, N_PAIRS, pair_body, 0)
    wait_out(ob_a, osem_a, N_CHUNKS - 2)
    wait_out(ob_b, osem_b, N_CHUNKS - 1)


def kernel(word_ids, modalities_ids, age_ids, delays_ids, seg_ids, posi_ids,
           NPI_ids, word_table, modalities_table, seg_table, NPI_table,
           posi_table, age_table, delay_table, ln_gamma, ln_beta):
    ids3 = jnp.stack([
        word_ids.reshape(-1), NPI_ids.reshape(-1),
        delays_ids.reshape(-1), posi_ids.reshape(-1),
        seg_ids.reshape(-1) * 10 + modalities_ids.reshape(-1),
        age_ids.reshape(-1),
    ])
    ids3 = ids3.reshape(6, BL // C, C).transpose(1, 0, 2)
    out = _embed_ln(
        ids3, word_table, NPI_table, delay_table, posi_table,
        seg_table, modalities_table, age_table, ln_gamma, ln_beta)
    return out.reshape(B, L, H)


# speedup vs baseline: 1.4496x; 1.4496x over previous
import functools

import jax
import jax.numpy as jnp
from jax import lax
from jax.experimental import pallas as pl
from jax.experimental.pallas import tpu as pltpu
from jax.experimental.pallas import tpu_sc as plsc

H = 128
B = 1024
L = 200
BL = B * L
EPS = 1e-12

NC = 2
NS = 16
NW = NC * NS
TOK_PER_W = BL // NW
C = 80
N_CHUNKS = TOK_PER_W // C
N_PAIRS = N_CHUNKS // 2
NV = H // 16


def _rsqrt(x):
    i = lax.bitcast_convert_type(x, jnp.int32)
    i = jnp.int32(0x5F3759DF) - lax.shift_right_logical(i, 1)
    y = lax.bitcast_convert_type(i, jnp.float32)
    for _ in range(3):
        y = y * (1.5 - 0.5 * x * y * y)
    return y


_GDN = lax.GatherDimensionNumbers(
    offset_dims=(), collapsed_slice_dims=(0,), start_index_map=(0,))


def _perm(v, idx):
    return lax.gather(v, idx[:, None], _GDN, (1,),
                      mode=lax.GatherScatterMode.PROMISE_IN_BOUNDS)


def _hsum(v):
    idx = lax.iota(jnp.int32, 16)
    for d in (8, 4, 2, 1):
        v = v + _perm(v, idx ^ d)
    return v


_MESH = plsc.VectorSubcoreMesh(
    core_axis_name="c", subcore_axis_name="s", num_cores=NC, num_subcores=NS
)


@functools.partial(
    pl.kernel,
    out_type=jax.ShapeDtypeStruct((BL, H), jnp.float32),
    mesh=_MESH,
    scratch_types=(
        [pltpu.VMEM((6, C), jnp.int32) for _ in range(2)]
        + [pltpu.VMEM((4, C, H), jnp.float32) for _ in range(2)]
        + [pltpu.VMEM((C, H), jnp.float32) for _ in range(2)]
        + [pltpu.VMEM((2, H), jnp.float32),
           pltpu.VMEM((10, H), jnp.float32),
           pltpu.VMEM((20, H), jnp.float32),
           pltpu.VMEM((120, H), jnp.float32),
           pltpu.VMEM((H,), jnp.float32), pltpu.VMEM((H,), jnp.float32),
           pltpu.SemaphoreType.DMA, pltpu.SemaphoreType.DMA,
           pltpu.SemaphoreType.DMA, pltpu.SemaphoreType.DMA]
    ),
)
def _embed_ln(ids3, wt, nt, dt, pt, st, mt, at, g, b,
              out,
              idx_a, idx_b, rows_a, rows_b, ob_a, ob_b,
              seg_v, mod_v, sm_v, age_v, gv, bv,
              sem_a, sem_b, osem_a, osem_b):
    wid = lax.axis_index("c") * NS + lax.axis_index("s")
    chunk0 = wid * N_CHUNKS
    tok0 = wid * TOK_PER_W

    pltpu.sync_copy(st, seg_v)
    pltpu.sync_copy(mt, mod_v)
    pltpu.sync_copy(at, age_v)
    pltpu.sync_copy(g, gv)
    pltpu.sync_copy(b, bv)
    gs = [gv[pl.ds(k * 16, 16)] for k in range(NV)]
    bs = [bv[pl.ds(k * 16, 16)] for k in range(NV)]

    def build_sm(sm, c2):
        s_i = sm // 10
        m_i = sm - s_i * 10
        for k in range(NV):
            sl = pl.ds(k * 16, 16)
            sm_v[sm, sl] = seg_v[s_i, sl] + mod_v[m_i, sl]
        return c2
    lax.fori_loop(0, 20, build_sm, 0)

    htabs = (wt, nt, dt, pt)
    rot8 = lax.iota(jnp.int32, 16) ^ 8

    def fire_gather(idx, rows, sem, ci):
        pltpu.sync_copy(ids3.at[chunk0 + ci], idx)
        for ti in range(4):
            pltpu.async_copy(htabs[ti].at[idx.at[ti]], rows.at[ti], sem)

    def wait_gather(idx, rows, sem):
        for ti in range(4):
            pltpu.make_async_copy(htabs[ti].at[idx.at[ti]], rows.at[ti],
                                  sem).wait()

    def compute_token(rows, ob, t, id_sm, id_a):
        vs = []
        for k in range(NV):
            sl = pl.ds(k * 16, 16)
            v = ((rows[0, t, sl] + rows[1, t, sl])
                 + (rows[2, t, sl] + rows[3, t, sl])
                 + (sm_v[id_sm, sl] + age_v[id_a, sl]))
            vs.append(v)
        s = ((vs[0] + vs[1]) + (vs[2] + vs[3])) + (
            (vs[4] + vs[5]) + (vs[6] + vs[7]))
        sq = ((vs[0] * vs[0] + vs[1] * vs[1])
              + (vs[2] * vs[2] + vs[3] * vs[3])) + (
             (vs[4] * vs[4] + vs[5] * vs[5])
              + (vs[6] * vs[6] + vs[7] * vs[7]))
        u = _hsum(s) * (1.0 / H)
        ex2 = _hsum(sq) * (1.0 / H)
        var = jnp.maximum(ex2 - u * u, 0.0)
        inv = _rsqrt(var + EPS)
        for k in range(NV):
            ob[t, pl.ds(k * 16, 16)] = (vs[k] - u) * inv * gs[k] + bs[k]

    def compute_chunk(idx, rows, ob, osem, ci):
        def group_body(g16, c2):
            base16 = g16 * 16
            ids_sm0 = idx[4, pl.ds(base16, 16)]
            ids_a0 = idx[5, pl.ds(base16, 16)]

            def half_body(hf, carry):
                ids_sm, ids_a = carry
                t = base16 + hf * 8
                for u in range(8):
                    compute_token(rows, ob, t + u, ids_sm[u], ids_a[u])
                return (_perm(ids_sm, rot8), _perm(ids_a, rot8))
            lax.fori_loop(0, 2, half_body, (ids_sm0, ids_a0))
            return c2
        lax.fori_loop(0, C // 16, group_body, 0)
        pltpu.async_copy(ob, out.at[pl.ds(tok0 + ci * C, C)], osem)

    def wait_out(ob, osem, ci):
        pltpu.make_async_copy(ob, out.at[pl.ds(tok0 + ci * C, C)],
                              osem).wait()

    fire_gather(idx_a, rows_a, sem_a, 0)

    def pair_body(p, carry):
        ca = 2 * p
        cb = 2 * p + 1
        fire_gather(idx_b, rows_b, sem_b, cb)
        wait_gather(idx_a, rows_a, sem_a)

        @pl.when(p > 0)
        def _():
            wait_out(ob_a, osem_a, ca - 2)
        compute_chunk(idx_a, rows_a, ob_a, osem_a, ca)

        @pl.when(p < N_PAIRS - 1)
        def _():
            fire_gather(idx_a, rows_a, sem_a, ca + 2)

        wait_gather(idx_b, rows_b, sem_b)

        @pl.when(p > 0)
        def _():
            wait_out(ob_b, osem_b, cb - 2)
        compute_chunk(idx_b, rows_b, ob_b, osem_b, cb)
        return carry

    lax.fori_loop(0, N_PAIRS, pair_body, 0)
    wait_out(ob_a, osem_a, N_CHUNKS - 2)
    wait_out(ob_b, osem_b, N_CHUNKS - 1)


def kernel(word_ids, modalities_ids, age_ids, delays_ids, seg_ids, posi_ids,
           NPI_ids, word_table, modalities_table, seg_table, NPI_table,
           posi_table, age_table, delay_table, ln_gamma, ln_beta):
    ids3 = jnp.stack([
        word_ids.reshape(-1), NPI_ids.reshape(-1),
        delays_ids.reshape(-1), posi_ids.reshape(-1),
        seg_ids.reshape(-1) * 10 + modalities_ids.reshape(-1),
        age_ids.reshape(-1),
    ])
    ids3 = ids3.reshape(6, BL // C, C).transpose(1, 0, 2)
    out = _embed_ln(
        ids3, word_table, NPI_table, delay_table, posi_table,
        seg_table, modalities_table, age_table, ln_gamma, ln_beta)
    return out.reshape(B, L, H)
